# baseline (speedup 1.0000x reference)
import jax
import jax.numpy as jnp
from jax import lax
from jax.experimental import pallas as pl
from jax.experimental.pallas import tpu as pltpu

_B, _C, _S, _N, _D = 8, 16, 4, 256, 16
_DLP, _DM, _NCLS = 16, 16, 2
_CC = 4
_NQ = _C // _CC
_NCHUNK = _B * _NQ
_NBUF = 3


def _vgl_body(adj_hbm, feat_hbm, wlp_ref, wm1_ref, wm2_ref, wdec_ref,
              bdec_ref, out_ref, abuf, fbuf, z_scr, asem, fsem):

    def adj_copy(i):
        b, q = divmod(i, _NQ)
        return pltpu.make_async_copy(
            adj_hbm.at[b, pl.ds(q * _CC, _CC)], abuf.at[i % _NBUF],
            asem.at[i % _NBUF])

    def feat_copy(b):
        return pltpu.make_async_copy(
            feat_hbm.at[b], fbuf.at[b % 2], fsem.at[b % 2])

    feat_copy(0).start()
    for j in range(_NBUF):
        adj_copy(j).start()

    for i in range(_NCHUNK):
        b, q = divmod(i, _NQ)
        if q == 0:
            feat_copy(b).wait()
        adj_copy(i).wait()

        for cl in range(_CC):
            cc = q * _CC + cl
            for s in range(_S):
                fwT = lax.dot_general(wlp_ref[cc, s], fbuf[b % 2, cc, s],
                                      (((0,), (1,)), ((), ())),
                                      preferred_element_type=jnp.float32)
                hT = jnp.maximum(
                    lax.dot_general(fwT, abuf[i % _NBUF, cl, s],
                                    (((1,), (1,)), ((), ())),
                                    preferred_element_type=jnp.float32),
                    0.0)
                z_scr[pl.ds(cc, 1), pl.ds(s * _DLP, _DLP), :] = hT[None]

        if i + _NBUF < _NCHUNK:
            adj_copy(i + _NBUF).start()
        if q == 0 and b + 1 < _B:
            feat_copy(b + 1).start()

        if q == _NQ - 1:
            z = z_scr[...].reshape(_C, _S * _DLP * _N)
            g = lax.dot_general(z, z, (((1,), (1,)), ((), ())),
                                preferred_element_type=jnp.float32)
            rows = lax.broadcasted_iota(jnp.int32, (_C, _C), 0)
            cols = lax.broadcasted_iota(jnp.int32, (_C, _C), 1)
            eye = (rows == cols).astype(jnp.float32)
            dcol = jnp.sum(g * eye, axis=1, keepdims=True)
            drow = jnp.sum(g * eye, axis=0, keepdims=True)
            denom = (jnp.sqrt(dcol) + 1e-8) * (jnp.sqrt(drow) + 1e-8)
            bg = g / denom
            h1 = jnp.maximum(
                jnp.dot(bg, wm1_ref[...],
                        preferred_element_type=jnp.float32),
                0.0)
            h2 = jnp.maximum(
                jnp.dot(bg, jnp.dot(h1, wm2_ref[...],
                                    preferred_element_type=jnp.float32),
                        preferred_element_type=jnp.float32),
                0.0)
            dec = jnp.dot(h2, wdec_ref[...],
                          preferred_element_type=jnp.float32) + bdec_ref[...]
            pooled = jnp.mean(dec, axis=0, keepdims=True)
            out_ref[b, pl.ds(0, 1), pl.ds(0, _NCLS)] = jax.nn.sigmoid(pooled)


def kernel(feats, adjs, W_lp, W_m1, W_m2, W_dec, b_dec):
    b_dec2 = b_dec.reshape(1, _NCLS)
    out = pl.pallas_call(
        _vgl_body,
        in_specs=[
            pl.BlockSpec(memory_space=pl.ANY),
            pl.BlockSpec(memory_space=pl.ANY),
            pl.BlockSpec(memory_space=pltpu.VMEM),
            pl.BlockSpec(memory_space=pltpu.VMEM),
            pl.BlockSpec(memory_space=pltpu.VMEM),
            pl.BlockSpec(memory_space=pltpu.VMEM),
            pl.BlockSpec(memory_space=pltpu.VMEM),
        ],
        out_specs=pl.BlockSpec(memory_space=pltpu.VMEM),
        out_shape=jax.ShapeDtypeStruct((_B, 8, 128), jnp.float32),
        scratch_shapes=[
            pltpu.VMEM((_NBUF, _CC, _S, _N, _N), jnp.float32),
            pltpu.VMEM((2, _C, _S, _N, _D), jnp.float32),
            pltpu.VMEM((_C, _S * _DLP, _N), jnp.float32),
            pltpu.SemaphoreType.DMA((_NBUF,)),
            pltpu.SemaphoreType.DMA((2,)),
        ],
    )(adjs, feats, W_lp, W_m1, W_m2, W_dec, b_dec2)
    return out[:, 0, :_NCLS]


# speedup vs baseline: 1.0643x; 1.0643x over previous
import jax
import jax.numpy as jnp
from jax import lax
from jax.experimental import pallas as pl
from jax.experimental.pallas import tpu as pltpu

_B, _C, _S, _N, _D = 8, 16, 4, 256, 16
_DLP, _DM, _NCLS = 16, 16, 2
_CC = 8
_NQ = _C // _CC
_NCHUNK = _B * _NQ
_NBUF = 3


def _vgl_body(adj_hbm, feat_hbm, wlp_ref, wm1_ref, wm2_ref, wdec_ref,
              bdec_ref, out_ref, abuf, fbuf, z_scr, asem, fsem):

    def adj_copy(i):
        b, q = divmod(i, _NQ)
        return pltpu.make_async_copy(
            adj_hbm.at[b, pl.ds(q * _CC, _CC)], abuf.at[i % _NBUF],
            asem.at[i % _NBUF])

    def feat_copy(b):
        return pltpu.make_async_copy(
            feat_hbm.at[b], fbuf.at[b % 2], fsem.at[b % 2])

    feat_copy(0).start()
    for j in range(_NBUF):
        adj_copy(j).start()

    for i in range(_NCHUNK):
        b, q = divmod(i, _NQ)
        if q == 0:
            feat_copy(b).wait()
        adj_copy(i).wait()

        for cl in range(_CC):
            cc = q * _CC + cl
            for s in range(_S):
                fwT = lax.dot_general(wlp_ref[cc, s], fbuf[b % 2, cc, s],
                                      (((0,), (1,)), ((), ())),
                                      preferred_element_type=jnp.float32)
                hT = jnp.maximum(
                    lax.dot_general(fwT, abuf[i % _NBUF, cl, s],
                                    (((1,), (1,)), ((), ())),
                                    preferred_element_type=jnp.float32),
                    0.0)
                z_scr[pl.ds(cc, 1), pl.ds(s * _DLP, _DLP), :] = hT[None]

        if i + _NBUF < _NCHUNK:
            adj_copy(i + _NBUF).start()
        if q == 0 and b + 1 < _B:
            feat_copy(b + 1).start()

        if q == _NQ - 1:
            z = z_scr[...].reshape(_C, _S * _DLP * _N)
            g = lax.dot_general(z, z, (((1,), (1,)), ((), ())),
                                preferred_element_type=jnp.float32)
            rows = lax.broadcasted_iota(jnp.int32, (_C, _C), 0)
            cols = lax.broadcasted_iota(jnp.int32, (_C, _C), 1)
            eye = (rows == cols).astype(jnp.float32)
            dcol = jnp.sum(g * eye, axis=1, keepdims=True)
            drow = jnp.sum(g * eye, axis=0, keepdims=True)
            denom = (jnp.sqrt(dcol) + 1e-8) * (jnp.sqrt(drow) + 1e-8)
            bg = g / denom
            h1 = jnp.maximum(
                jnp.dot(bg, wm1_ref[...],
                        preferred_element_type=jnp.float32),
                0.0)
            h2 = jnp.maximum(
                jnp.dot(bg, jnp.dot(h1, wm2_ref[...],
                                    preferred_element_type=jnp.float32),
                        preferred_element_type=jnp.float32),
                0.0)
            dec = jnp.dot(h2, wdec_ref[...],
                          preferred_element_type=jnp.float32) + bdec_ref[...]
            pooled = jnp.mean(dec, axis=0, keepdims=True)
            out_ref[b, pl.ds(0, 1), pl.ds(0, _NCLS)] = jax.nn.sigmoid(pooled)


def kernel(feats, adjs, W_lp, W_m1, W_m2, W_dec, b_dec):
    b_dec2 = b_dec.reshape(1, _NCLS)
    out = pl.pallas_call(
        _vgl_body,
        in_specs=[
            pl.BlockSpec(memory_space=pl.ANY),
            pl.BlockSpec(memory_space=pl.ANY),
            pl.BlockSpec(memory_space=pltpu.VMEM),
            pl.BlockSpec(memory_space=pltpu.VMEM),
            pl.BlockSpec(memory_space=pltpu.VMEM),
            pl.BlockSpec(memory_space=pltpu.VMEM),
            pl.BlockSpec(memory_space=pltpu.VMEM),
        ],
        out_specs=pl.BlockSpec(memory_space=pltpu.VMEM),
        out_shape=jax.ShapeDtypeStruct((_B, 8, 128), jnp.float32),
        scratch_shapes=[
            pltpu.VMEM((_NBUF, _CC, _S, _N, _N), jnp.float32),
            pltpu.VMEM((2, _C, _S, _N, _D), jnp.float32),
            pltpu.VMEM((_C, _S * _DLP, _N), jnp.float32),
            pltpu.SemaphoreType.DMA((_NBUF,)),
            pltpu.SemaphoreType.DMA((2,)),
        ],
    )(adjs, feats, W_lp, W_m1, W_m2, W_dec, b_dec2)
    return out[:, 0, :_NCLS]
